# baseline (speedup 1.0000x reference)
import functools

import jax
import jax.numpy as jnp
from jax import lax
from jax.experimental import pallas as pl
from jax.experimental.pallas import tpu as pltpu
from jax.experimental.pallas import tpu_sc as plsc

N, D, H, E, K = 2048, 768, 64, 64, 2
BS = 128
NBLK = H + N * K // BS
PT = NBLK * BS
CH = N // BS
EPS = 1e-6
NEG = -1e30

NC, NS = 2, 16
NW = NC * NS
TPW = N // NW


def _routing_body(x_ref, wsel_ref, pos1_ref, pos2_ref, g1_ref, g2_ref,
                  hb_ref, vd_ref, fr_ref, nr_ref, c_scr, s_scr):
    x = x_ref[...]
    logits = jnp.dot(x, wsel_ref[...], preferred_element_type=jnp.float32)
    ii = lax.broadcasted_iota(jnp.int32, (N, H), 1)
    m1 = jnp.max(logits, axis=1, keepdims=True)
    i1 = jnp.min(jnp.where(logits == m1, ii, H), axis=1, keepdims=True)
    l2 = jnp.where(ii == i1, NEG, logits)
    m2 = jnp.max(l2, axis=1, keepdims=True)
    i2 = jnp.min(jnp.where(l2 == m2, ii, H), axis=1, keepdims=True)
    z = jnp.sum(jnp.exp(logits - m1), axis=1, keepdims=True)
    p1 = 1.0 / z
    p2 = jnp.exp(m2 - m1) / z
    s = p1 + p2 + EPS
    ones16 = jnp.ones((1, 128), jnp.float32)
    g1_ref[...] = (p1 / s) * ones16
    g2_ref[...] = (p2 / s) * ones16

    oh1 = (ii == i1).astype(jnp.float32)
    oh2 = (ii == i2).astype(jnp.float32)
    c = oh1 + oh2
    c_scr[...] = c
    cnt = jnp.sum(c, axis=0, keepdims=True)
    nb = jnp.maximum(jnp.ceil(cnt * (1.0 / BS)), 1.0)
    hh0 = lax.broadcasted_iota(jnp.int32, (H, H), 0)
    hh1 = lax.broadcasted_iota(jnp.int32, (H, H), 1)
    upper = (hh0 < hh1).astype(jnp.float32)
    bsf = jnp.dot(nb, upper, preferred_element_type=jnp.float32)

    rr0 = lax.broadcasted_iota(jnp.int32, (BS, BS), 0)
    rr1 = lax.broadcasted_iota(jnp.int32, (BS, BS), 1)
    tril = (rr1 < rr0).astype(jnp.float32)

    def chunk(cix, base):
        blk = c_scr[pl.ds(cix * BS, BS), :]
        s_scr[pl.ds(cix * BS, BS), :] = (
            jnp.dot(tril, blk, preferred_element_type=jnp.float32) + base)
        return base + jnp.sum(blk, axis=0, keepdims=True)

    lax.fori_loop(0, CH, chunk, jnp.zeros((1, H), jnp.float32))
    srank = s_scr[...]
    rank1 = jnp.sum(srank * oh1, axis=1, keepdims=True)
    rank2 = jnp.sum(srank * oh2, axis=1, keepdims=True)
    bs1 = jnp.sum(bsf * oh1, axis=1, keepdims=True)
    bs2 = jnp.sum(bsf * oh2, axis=1, keepdims=True)
    pos1_ref[...] = (bs1 * BS + rank1).astype(jnp.int32)
    pos2_ref[...] = (bs2 * BS + rank2).astype(jnp.int32)

    bb = lax.broadcasted_iota(jnp.int32, (NBLK, H), 0).astype(jnp.float32)
    hb = jnp.sum((bsf <= bb).astype(jnp.float32), axis=1, keepdims=True) - 1.0
    bh = lax.broadcasted_iota(jnp.int32, (NBLK, H), 1).astype(jnp.float32)
    ohb = (bh == hb).astype(jnp.float32)
    bs_at = jnp.sum(bsf * ohb, axis=1, keepdims=True)
    cnt_at = jnp.sum(cnt * ohb, axis=1, keepdims=True)
    bvals = bb[:, :1]
    base_rows = (bvals - bs_at) * BS
    vd = jnp.clip(cnt_at - base_rows, 0.0, float(BS))
    hb_ref[...] = hb.astype(jnp.int32)
    vd_ref[...] = vd.astype(jnp.int32)
    fr_ref[...] = (bvals == bs_at).astype(jnp.int32)
    nr_ref[...] = jnp.sum(nb, axis=1, keepdims=True).astype(jnp.int32)


def _routing(x, w_sel):
    outs = pl.pallas_call(
        _routing_body,
        grid=(1,),
        in_specs=[pl.BlockSpec((N, D), lambda i: (0, 0)),
                  pl.BlockSpec((D, H), lambda i: (0, 0))],
        out_specs=[pl.BlockSpec((N, 1), lambda i: (0, 0))] * 2
        + [pl.BlockSpec((N, 128), lambda i: (0, 0))] * 2
        + [pl.BlockSpec((NBLK, 1), lambda i: (0, 0))] * 3
        + [pl.BlockSpec((1, 1), lambda i: (0, 0))],
        out_shape=[jax.ShapeDtypeStruct((N, 1), jnp.int32),
                   jax.ShapeDtypeStruct((N, 1), jnp.int32),
                   jax.ShapeDtypeStruct((N, 128), jnp.float32),
                   jax.ShapeDtypeStruct((N, 128), jnp.float32),
                   jax.ShapeDtypeStruct((NBLK, 1), jnp.int32),
                   jax.ShapeDtypeStruct((NBLK, 1), jnp.int32),
                   jax.ShapeDtypeStruct((NBLK, 1), jnp.int32),
                   jax.ShapeDtypeStruct((1, 1), jnp.int32)],
        scratch_shapes=[pltpu.VMEM((N, H), jnp.float32),
                        pltpu.VMEM((N, H), jnp.float32)],
    )(x, w_sel)
    return outs


def _scatter_body(x_hbm, p1_hbm, p2_hbm, g1_hbm, g2_hbm, xg_hbm, gg_hbm,
                  xtok, idxa, idxb, gba, gbb, sem):
    wid = lax.axis_index("s") * NC + lax.axis_index("c")
    base = wid * TPW
    loads = [pltpu.async_copy(p1_hbm.at[pl.ds(base, TPW)], idxa, sem),
             pltpu.async_copy(p2_hbm.at[pl.ds(base, TPW)], idxb, sem),
             pltpu.async_copy(x_hbm.at[pl.ds(base, TPW)], xtok, sem),
             pltpu.async_copy(g1_hbm.at[pl.ds(base, TPW)], gba, sem),
             pltpu.async_copy(g2_hbm.at[pl.ds(base, TPW)], gbb, sem)]
    for ld in loads:
        ld.wait()
    stores = [pltpu.async_copy(xtok, xg_hbm.at[idxa], sem),
              pltpu.async_copy(xtok, xg_hbm.at[idxb], sem),
              pltpu.async_copy(gba, gg_hbm.at[idxa], sem),
              pltpu.async_copy(gbb, gg_hbm.at[idxb], sem)]
    for st in stores:
        st.wait()


def _scatter(x, p1, p2, g1w, g2w):
    mesh = plsc.VectorSubcoreMesh(core_axis_name="c", subcore_axis_name="s")
    f = pl.kernel(
        _scatter_body,
        out_type=[jax.ShapeDtypeStruct((PT, D), jnp.float32),
                  jax.ShapeDtypeStruct((PT, 128), jnp.float32)],
        mesh=mesh,
        scratch_types=[pltpu.VMEM((TPW, D), jnp.float32),
                       pltpu.VMEM((TPW,), jnp.int32),
                       pltpu.VMEM((TPW,), jnp.int32),
                       pltpu.VMEM((TPW, 128), jnp.float32),
                       pltpu.VMEM((TPW, 128), jnp.float32),
                       pltpu.SemaphoreType.DMA],
    )
    return f(x, p1, p2, g1w, g2w)


SUB = 8
NBJ = NBLK // SUB


def _passes_body(*refs):
    hb_ref, vd_ref, fr_ref, nr_ref, xg_ref, gg_ref = refs[:6]
    wa = refs[6:6 + 8 * SUB]
    wb = refs[6 + 8 * SUB:6 + 10 * SUB]
    (nkv_ref, nnm_ref, yg_ref, qg_scr, gs_scr, kv_scr,
     nm_scr) = refs[6 + 10 * SUB:]
    p = pl.program_id(0)
    j = pl.program_id(1)
    nr = nr_ref[0]
    rows = lax.broadcasted_iota(jnp.int32, (BS, 1), 0)
    live0 = SUB * j < nr

    @pl.when((p == 0) & live0)
    def _pa():
        for i in range(SUB):
            wq_ref, bq_ref, wk_ref, bk_ref, wv_ref, bv_ref, mkv_ref, \
                mnm_ref = wa[8 * i:8 * i + 8]
            b = SUB * j + i
            h = hb_ref[b]
            rmask = rows < vd_ref[b]
            xb = jnp.where(rmask, xg_ref[pl.ds(i * BS, BS), :], 0.0)
            g = jnp.where(rmask, gg_ref[pl.ds(i * BS, BS), 0:1], 0.0)
            q = jnp.dot(xb, wq_ref[0], preferred_element_type=jnp.float32) + bq_ref[0]
            k = jnp.dot(xb, wk_ref[0], preferred_element_type=jnp.float32) + bk_ref[0]
            v = jnp.dot(xb, wv_ref[0], preferred_element_type=jnp.float32) + bv_ref[0]
            qg_scr[pl.ds(b * BS, BS), :] = q
            gs_scr[pl.ds(b * BS, BS), :] = g
            kg = k * g
            kvc = lax.dot_general(kg, v, (((0,), (0,)), ((), ())),
                                  preferred_element_type=jnp.float32)
            nmc = jnp.sum(kg, axis=0, keepdims=True)

            @pl.when(fr_ref[b] == 1)
            def _init():
                kv_scr[pl.ds(h, 1)] = mkv_ref[...] + kvc[None]
                nm_scr[pl.ds(h, 1), :] = mnm_ref[0] + nmc

            @pl.when(fr_ref[b] == 0)
            def _acc():
                kv_scr[pl.ds(h, 1)] += kvc[None]
                nm_scr[pl.ds(h, 1), :] += nmc

    @pl.when((p == 1) & live0)
    def _pb():
        for i in range(SUB):
            wo_ref, bo_ref = wb[2 * i:2 * i + 2]
            b = SUB * j + i
            h = hb_ref[b]
            q = qg_scr[pl.ds(b * BS, BS), :]
            g = gs_scr[pl.ds(b * BS, BS), :]
            nkv = kv_scr[pl.ds(h, 1)][0]
            nnm = nm_scr[pl.ds(h, 1), :]
            num = jnp.dot(q, nkv, preferred_element_type=jnp.float32)
            den = jnp.sum(q * nnm, axis=1, keepdims=True) + EPS
            attn_g = jnp.where(g != 0.0, num / den * g, 0.0)
            yg = jnp.dot(attn_g, wo_ref[0], preferred_element_type=jnp.float32)
            yg_ref[pl.ds(i * BS, BS), :] = yg + g * bo_ref[0]

    @pl.when((p == 1) & (j == NBJ - 1))
    def _emit():
        nkv_ref[...] = kv_scr[...]
        nnm_ref[...] = nm_scr[...][:, None, :]


def _passes(hb, vd, fr, nr, xg, gg2, wq, bq3, wk, bk3, wv, bv3, mkv, mnm3, wo, bo3):
    def a_map(i):
        return lambda p, j, hb, vd, fr, nr, i=i: (
            jnp.where(p == 0, hb[SUB * j + i], 0), 0, 0)

    def b_map(i):
        return lambda p, j, hb, vd, fr, nr, i=i: (
            jnp.where(p == 1, hb[SUB * j + i], 0), 0, 0)

    aspecs = []
    for i in range(SUB):
        aspecs += [pl.BlockSpec((1, D, E), a_map(i)),
                   pl.BlockSpec((1, 1, E), a_map(i)),
                   pl.BlockSpec((1, D, E), a_map(i)),
                   pl.BlockSpec((1, 1, E), a_map(i)),
                   pl.BlockSpec((1, D, E), a_map(i)),
                   pl.BlockSpec((1, 1, E), a_map(i)),
                   pl.BlockSpec((1, E, E), a_map(i)),
                   pl.BlockSpec((1, 1, E), a_map(i))]
    bspecs = []
    for i in range(SUB):
        bspecs += [pl.BlockSpec((1, E, D), b_map(i)),
                   pl.BlockSpec((1, 1, D), b_map(i))]
    a_args = []
    for i in range(SUB):
        a_args += [wq, bq3, wk, bk3, wv, bv3, mkv, mnm3]
    b_args = []
    for i in range(SUB):
        b_args += [wo, bo3]
    return pl.pallas_call(
        _passes_body,
        grid_spec=pltpu.PrefetchScalarGridSpec(
            num_scalar_prefetch=4,
            grid=(2, NBJ),
            in_specs=[
                pl.BlockSpec((SUB * BS, D),
                             lambda p, j, hb, vd, fr, nr:
                             (jnp.where((p == 0) & (SUB * j < nr[0]), j, 0), 0)),
                pl.BlockSpec((SUB * BS, 128),
                             lambda p, j, hb, vd, fr, nr:
                             (jnp.where((p == 0) & (SUB * j < nr[0]), j, 0), 0)),
            ] + aspecs + bspecs,
            out_specs=[
                pl.BlockSpec((H, E, E), lambda p, j, hb, vd, fr, nr: (0, 0, 0)),
                pl.BlockSpec((H, 1, E), lambda p, j, hb, vd, fr, nr: (0, 0, 0)),
                pl.BlockSpec((SUB * BS, D),
                             lambda p, j, hb, vd, fr, nr:
                             (jnp.where((p == 1) & (SUB * j < nr[0]), j, NBJ - 1), 0)),
            ],
            scratch_shapes=[
                pltpu.VMEM((PT, E), jnp.float32),
                pltpu.VMEM((PT, 1), jnp.float32),
                pltpu.VMEM((H, E, E), jnp.float32),
                pltpu.VMEM((H, E), jnp.float32),
            ],
        ),
        out_shape=[jax.ShapeDtypeStruct((H, E, E), jnp.float32),
                   jax.ShapeDtypeStruct((H, 1, E), jnp.float32),
                   jax.ShapeDtypeStruct((PT, D), jnp.float32)],
    )(hb, vd, fr, nr, xg, gg2, *a_args, *b_args)


def _combine_body(yg_hbm, p1_hbm, p2_hbm, out_hbm, idxa, idxb, buf0, buf1, sem):
    wid = lax.axis_index("s") * NC + lax.axis_index("c")
    base = wid * TPW
    la = pltpu.async_copy(p1_hbm.at[pl.ds(base, TPW)], idxa, sem)
    lb = pltpu.async_copy(p2_hbm.at[pl.ds(base, TPW)], idxb, sem)
    la.wait()
    lb.wait()
    ga = pltpu.async_copy(yg_hbm.at[idxa], buf0, sem)
    gb = pltpu.async_copy(yg_hbm.at[idxb], buf1, sem)
    ga.wait()
    gb.wait()

    @plsc.parallel_loop(0, TPW * D // 16, unroll=8)
    def _add(j):
        r = j // (D // 16)
        sl = pl.ds((j % (D // 16)) * 16, 16)
        buf0[r, sl] = buf0[r, sl] + buf1[r, sl]

    pltpu.sync_copy(buf0, out_hbm.at[pl.ds(base, TPW)])


def _combine(yg, p1, p2):
    mesh = plsc.VectorSubcoreMesh(core_axis_name="c", subcore_axis_name="s")
    f = pl.kernel(
        _combine_body,
        out_type=[jax.ShapeDtypeStruct((N, D), jnp.float32)],
        mesh=mesh,
        scratch_types=[pltpu.VMEM((TPW,), jnp.int32),
                       pltpu.VMEM((TPW,), jnp.int32),
                       pltpu.VMEM((TPW, D), jnp.float32),
                       pltpu.VMEM((TPW, D), jnp.float32),
                       pltpu.SemaphoreType.DMA],
    )
    return f(yg, p1, p2)[0]


@jax.jit
def kernel(queries, mem_kv, mem_norm, w_sel, wq, bq, wk, bk, wv, bv, wo, bo):
    p1, p2, g1, g2, hb, vd, fr, nr = _routing(queries, w_sel)
    p1f, p2f = p1.reshape(N), p2.reshape(N)
    xg, gg = _scatter(queries, p1f, p2f, g1, g2)
    hbf, vdf, frf = hb.reshape(NBLK), vd.reshape(NBLK), fr.reshape(NBLK)
    nkv, nnm, yg = _passes(hbf, vdf, frf, nr.reshape(1), xg, gg,
                           wq, bq.reshape(H, 1, E), wk, bk.reshape(H, 1, E),
                           wv, bv.reshape(H, 1, E), mem_kv,
                           mem_norm.reshape(H, 1, E), wo, bo.reshape(H, 1, D))
    out = _combine(yg, p1f, p2f)
    return out, nkv, nnm.reshape(H, E)


# speedup vs baseline: 1.0106x; 1.0106x over previous
import jax
import jax.numpy as jnp
from jax import lax
from jax.experimental import pallas as pl
from jax.experimental.pallas import tpu as pltpu
from jax.experimental.pallas import tpu_sc as plsc

N, D, H, E, K = 2048, 768, 64, 64, 2
BS = 128
NBLK = H + N * K // BS
PT = NBLK * BS
CH = N // BS
EPS = 1e-6
NEG = -1e30

NC, NS = 2, 16
NW = NC * NS
TPW = N // NW


def _routing_body(x_ref, wsel_ref, pos1_ref, pos2_ref, g1_ref, g2_ref,
                  hb_ref, vd_ref, fr_ref, nr_ref, c_scr, s_scr):
    x = x_ref[...]
    logits = jnp.dot(x, wsel_ref[...], preferred_element_type=jnp.float32)
    ii = lax.broadcasted_iota(jnp.int32, (N, H), 1)
    m1 = jnp.max(logits, axis=1, keepdims=True)
    i1 = jnp.min(jnp.where(logits == m1, ii, H), axis=1, keepdims=True)
    l2 = jnp.where(ii == i1, NEG, logits)
    m2 = jnp.max(l2, axis=1, keepdims=True)
    i2 = jnp.min(jnp.where(l2 == m2, ii, H), axis=1, keepdims=True)
    z = jnp.sum(jnp.exp(logits - m1), axis=1, keepdims=True)
    p1 = 1.0 / z
    p2 = jnp.exp(m2 - m1) / z
    s = p1 + p2 + EPS
    ones16 = jnp.ones((1, 128), jnp.float32)
    g1_ref[...] = (p1 / s) * ones16
    g2_ref[...] = (p2 / s) * ones16

    oh1 = (ii == i1).astype(jnp.float32)
    oh2 = (ii == i2).astype(jnp.float32)
    c = oh1 + oh2
    c_scr[...] = c
    cnt = jnp.sum(c, axis=0, keepdims=True)
    nb = jnp.maximum(jnp.ceil(cnt * (1.0 / BS)), 1.0)
    hh0 = lax.broadcasted_iota(jnp.int32, (H, H), 0)
    hh1 = lax.broadcasted_iota(jnp.int32, (H, H), 1)
    upper = (hh0 < hh1).astype(jnp.float32)
    bsf = jnp.dot(nb, upper, preferred_element_type=jnp.float32)

    rr0 = lax.broadcasted_iota(jnp.int32, (BS, BS), 0)
    rr1 = lax.broadcasted_iota(jnp.int32, (BS, BS), 1)
    tril = (rr1 < rr0).astype(jnp.float32)

    def chunk(cix, base):
        blk = c_scr[pl.ds(cix * BS, BS), :]
        s_scr[pl.ds(cix * BS, BS), :] = (
            jnp.dot(tril, blk, preferred_element_type=jnp.float32) + base)
        return base + jnp.sum(blk, axis=0, keepdims=True)

    lax.fori_loop(0, CH, chunk, jnp.zeros((1, H), jnp.float32))
    srank = s_scr[...]
    rank1 = jnp.sum(srank * oh1, axis=1, keepdims=True)
    rank2 = jnp.sum(srank * oh2, axis=1, keepdims=True)
    bs1 = jnp.sum(bsf * oh1, axis=1, keepdims=True)
    bs2 = jnp.sum(bsf * oh2, axis=1, keepdims=True)
    pos1_ref[...] = (bs1 * BS + rank1).astype(jnp.int32)
    pos2_ref[...] = (bs2 * BS + rank2).astype(jnp.int32)

    bb = lax.broadcasted_iota(jnp.int32, (NBLK, H), 0).astype(jnp.float32)
    hb = jnp.sum((bsf <= bb).astype(jnp.float32), axis=1, keepdims=True) - 1.0
    bh = lax.broadcasted_iota(jnp.int32, (NBLK, H), 1).astype(jnp.float32)
    ohb = (bh == hb).astype(jnp.float32)
    bs_at = jnp.sum(bsf * ohb, axis=1, keepdims=True)
    cnt_at = jnp.sum(cnt * ohb, axis=1, keepdims=True)
    bvals = bb[:, :1]
    base_rows = (bvals - bs_at) * BS
    vd = jnp.clip(cnt_at - base_rows, 0.0, float(BS))
    hb_ref[...] = hb.astype(jnp.int32)
    vd_ref[...] = vd.astype(jnp.int32)
    fr_ref[...] = (bvals == bs_at).astype(jnp.int32)
    nr_ref[...] = jnp.sum(nb, axis=1, keepdims=True).astype(jnp.int32)


def _routing(x, w_sel):
    outs = pl.pallas_call(
        _routing_body,
        grid=(1,),
        in_specs=[pl.BlockSpec((N, D), lambda i: (0, 0)),
                  pl.BlockSpec((D, H), lambda i: (0, 0))],
        out_specs=[pl.BlockSpec((N, 1), lambda i: (0, 0))] * 2
        + [pl.BlockSpec((N, 128), lambda i: (0, 0))] * 2
        + [pl.BlockSpec((NBLK, 1), lambda i: (0, 0))] * 3
        + [pl.BlockSpec((1, 1), lambda i: (0, 0))],
        out_shape=[jax.ShapeDtypeStruct((N, 1), jnp.int32),
                   jax.ShapeDtypeStruct((N, 1), jnp.int32),
                   jax.ShapeDtypeStruct((N, 128), jnp.float32),
                   jax.ShapeDtypeStruct((N, 128), jnp.float32),
                   jax.ShapeDtypeStruct((NBLK, 1), jnp.int32),
                   jax.ShapeDtypeStruct((NBLK, 1), jnp.int32),
                   jax.ShapeDtypeStruct((NBLK, 1), jnp.int32),
                   jax.ShapeDtypeStruct((1, 1), jnp.int32)],
        scratch_shapes=[pltpu.VMEM((N, H), jnp.float32),
                        pltpu.VMEM((N, H), jnp.float32)],
    )(x, w_sel)
    return outs


def _scatter_body(x_hbm, p1_hbm, p2_hbm, g1_hbm, g2_hbm, xg_hbm, gg_hbm,
                  xtok, idxa, idxb, gba, gbb, sem):
    wid = lax.axis_index("s") * NC + lax.axis_index("c")
    base = wid * TPW
    loads = [pltpu.async_copy(p1_hbm.at[pl.ds(base, TPW)], idxa, sem),
             pltpu.async_copy(p2_hbm.at[pl.ds(base, TPW)], idxb, sem),
             pltpu.async_copy(x_hbm.at[pl.ds(base, TPW)], xtok, sem),
             pltpu.async_copy(g1_hbm.at[pl.ds(base, TPW)], gba, sem),
             pltpu.async_copy(g2_hbm.at[pl.ds(base, TPW)], gbb, sem)]
    for ld in loads:
        ld.wait()
    stores = [pltpu.async_copy(xtok, xg_hbm.at[idxa], sem),
              pltpu.async_copy(xtok, xg_hbm.at[idxb], sem),
              pltpu.async_copy(gba, gg_hbm.at[idxa], sem),
              pltpu.async_copy(gbb, gg_hbm.at[idxb], sem)]
    for st in stores:
        st.wait()


def _scatter(x, p1, p2, g1w, g2w):
    mesh = plsc.VectorSubcoreMesh(core_axis_name="c", subcore_axis_name="s")
    f = pl.kernel(
        _scatter_body,
        out_type=[jax.ShapeDtypeStruct((PT, D), jnp.float32),
                  jax.ShapeDtypeStruct((PT, 128), jnp.float32)],
        mesh=mesh,
        scratch_types=[pltpu.VMEM((TPW, D), jnp.float32),
                       pltpu.VMEM((TPW,), jnp.int32),
                       pltpu.VMEM((TPW,), jnp.int32),
                       pltpu.VMEM((TPW, 128), jnp.float32),
                       pltpu.VMEM((TPW, 128), jnp.float32),
                       pltpu.SemaphoreType.DMA],
    )
    return f(x, p1, p2, g1w, g2w)


SUB = 4
NBJ = NBLK // SUB


def _passes_body(*refs):
    hb_ref, vd_ref, fr_ref, nr_ref, xg_ref, gg_ref = refs[:6]
    wa = refs[6:6 + 8 * SUB]
    wb = refs[6 + 8 * SUB:6 + 10 * SUB]
    (nkv_ref, nnm_ref, yg_ref, qg_scr, gs_scr, kv_scr,
     nm_scr) = refs[6 + 10 * SUB:]
    p = pl.program_id(0)
    j = pl.program_id(1)
    nr = nr_ref[0]
    rows = lax.broadcasted_iota(jnp.int32, (BS, 1), 0)
    live0 = SUB * j < nr

    @pl.when((p == 0) & live0)
    def _pa():
        for i in range(SUB):
            wq_ref, bq_ref, wk_ref, bk_ref, wv_ref, bv_ref, mkv_ref, \
                mnm_ref = wa[8 * i:8 * i + 8]
            b = SUB * j + i
            h = hb_ref[b]
            rmask = rows < vd_ref[b]
            xb = jnp.where(rmask, xg_ref[pl.ds(i * BS, BS), :], 0.0)
            g = jnp.where(rmask, gg_ref[pl.ds(i * BS, BS), 0:1], 0.0)
            q = jnp.dot(xb, wq_ref[0], preferred_element_type=jnp.float32) + bq_ref[0]
            k = jnp.dot(xb, wk_ref[0], preferred_element_type=jnp.float32) + bk_ref[0]
            v = jnp.dot(xb, wv_ref[0], preferred_element_type=jnp.float32) + bv_ref[0]
            qg_scr[pl.ds(b * BS, BS), :] = q
            gs_scr[pl.ds(b * BS, BS), :] = g
            kg = k * g
            kvc = lax.dot_general(kg, v, (((0,), (0,)), ((), ())),
                                  preferred_element_type=jnp.float32)
            nmc = jnp.sum(kg, axis=0, keepdims=True)

            @pl.when(fr_ref[b] == 1)
            def _init():
                kv_scr[pl.ds(h, 1)] = mkv_ref[...] + kvc[None]
                nm_scr[pl.ds(h, 1), :] = mnm_ref[0] + nmc

            @pl.when(fr_ref[b] == 0)
            def _acc():
                kv_scr[pl.ds(h, 1)] += kvc[None]
                nm_scr[pl.ds(h, 1), :] += nmc

    @pl.when((p == 1) & live0)
    def _pb():
        for i in range(SUB):
            wo_ref, bo_ref = wb[2 * i:2 * i + 2]
            b = SUB * j + i
            h = hb_ref[b]
            q = qg_scr[pl.ds(b * BS, BS), :]
            g = gs_scr[pl.ds(b * BS, BS), :]
            nkv = kv_scr[pl.ds(h, 1)][0]
            nnm = nm_scr[pl.ds(h, 1), :]
            num = jnp.dot(q, nkv, preferred_element_type=jnp.float32)
            den = jnp.sum(q * nnm, axis=1, keepdims=True) + EPS
            attn_g = jnp.where(g != 0.0, num / den * g, 0.0)
            yg = jnp.dot(attn_g, wo_ref[0], preferred_element_type=jnp.float32)
            yg_ref[pl.ds(i * BS, BS), :] = yg + g * bo_ref[0]

    @pl.when((p == 1) & (j == NBJ - 1))
    def _emit():
        nkv_ref[...] = kv_scr[...]
        nnm_ref[...] = nm_scr[...][:, None, :]


def _passes(hb, vd, fr, nr, xg, gg2, wq, bq3, wk, bk3, wv, bv3, mkv, mnm3, wo, bo3):
    def a_map(i):
        return lambda p, j, hb, vd, fr, nr, i=i: (
            jnp.where(p == 0, hb[SUB * j + i], 0), 0, 0)

    def b_map(i):
        return lambda p, j, hb, vd, fr, nr, i=i: (
            jnp.where(p == 1, hb[SUB * j + i], 0), 0, 0)

    aspecs = []
    for i in range(SUB):
        aspecs += [pl.BlockSpec((1, D, E), a_map(i)),
                   pl.BlockSpec((1, 1, E), a_map(i)),
                   pl.BlockSpec((1, D, E), a_map(i)),
                   pl.BlockSpec((1, 1, E), a_map(i)),
                   pl.BlockSpec((1, D, E), a_map(i)),
                   pl.BlockSpec((1, 1, E), a_map(i)),
                   pl.BlockSpec((1, E, E), a_map(i)),
                   pl.BlockSpec((1, 1, E), a_map(i))]
    bspecs = []
    for i in range(SUB):
        bspecs += [pl.BlockSpec((1, E, D), b_map(i)),
                   pl.BlockSpec((1, 1, D), b_map(i))]
    a_args = []
    for i in range(SUB):
        a_args += [wq, bq3, wk, bk3, wv, bv3, mkv, mnm3]
    b_args = []
    for i in range(SUB):
        b_args += [wo, bo3]
    return pl.pallas_call(
        _passes_body,
        grid_spec=pltpu.PrefetchScalarGridSpec(
            num_scalar_prefetch=4,
            grid=(2, NBJ),
            in_specs=[
                pl.BlockSpec((SUB * BS, D),
                             lambda p, j, hb, vd, fr, nr:
                             (jnp.where((p == 0) & (SUB * j < nr[0]), j, 0), 0)),
                pl.BlockSpec((SUB * BS, 128),
                             lambda p, j, hb, vd, fr, nr:
                             (jnp.where((p == 0) & (SUB * j < nr[0]), j, 0), 0)),
            ] + aspecs + bspecs,
            out_specs=[
                pl.BlockSpec((H, E, E), lambda p, j, hb, vd, fr, nr: (0, 0, 0)),
                pl.BlockSpec((H, 1, E), lambda p, j, hb, vd, fr, nr: (0, 0, 0)),
                pl.BlockSpec((SUB * BS, D),
                             lambda p, j, hb, vd, fr, nr:
                             (jnp.where((p == 1) & (SUB * j < nr[0]), j, NBJ - 1), 0)),
            ],
            scratch_shapes=[
                pltpu.VMEM((PT, E), jnp.float32),
                pltpu.VMEM((PT, 1), jnp.float32),
                pltpu.VMEM((H, E, E), jnp.float32),
                pltpu.VMEM((H, E), jnp.float32),
            ],
        ),
        out_shape=[jax.ShapeDtypeStruct((H, E, E), jnp.float32),
                   jax.ShapeDtypeStruct((H, 1, E), jnp.float32),
                   jax.ShapeDtypeStruct((PT, D), jnp.float32)],
    )(hb, vd, fr, nr, xg, gg2, *a_args, *b_args)


def _combine_body(yg_hbm, p1_hbm, p2_hbm, out_hbm, idxa, idxb, buf0, buf1, sem):
    wid = lax.axis_index("s") * NC + lax.axis_index("c")
    base = wid * TPW
    la = pltpu.async_copy(p1_hbm.at[pl.ds(base, TPW)], idxa, sem)
    lb = pltpu.async_copy(p2_hbm.at[pl.ds(base, TPW)], idxb, sem)
    la.wait()
    lb.wait()
    ga = pltpu.async_copy(yg_hbm.at[idxa], buf0, sem)
    gb = pltpu.async_copy(yg_hbm.at[idxb], buf1, sem)
    ga.wait()
    gb.wait()

    @plsc.parallel_loop(0, TPW * D // 16, unroll=8)
    def _add(j):
        r = j // (D // 16)
        sl = pl.ds((j % (D // 16)) * 16, 16)
        buf0[r, sl] = buf0[r, sl] + buf1[r, sl]

    pltpu.sync_copy(buf0, out_hbm.at[pl.ds(base, TPW)])


def _combine(yg, p1, p2):
    mesh = plsc.VectorSubcoreMesh(core_axis_name="c", subcore_axis_name="s")
    f = pl.kernel(
        _combine_body,
        out_type=[jax.ShapeDtypeStruct((N, D), jnp.float32)],
        mesh=mesh,
        scratch_types=[pltpu.VMEM((TPW,), jnp.int32),
                       pltpu.VMEM((TPW,), jnp.int32),
                       pltpu.VMEM((TPW, D), jnp.float32),
                       pltpu.VMEM((TPW, D), jnp.float32),
                       pltpu.SemaphoreType.DMA],
    )
    return f(yg, p1, p2)[0]


@jax.jit
def kernel(queries, mem_kv, mem_norm, w_sel, wq, bq, wk, bk, wv, bv, wo, bo):
    p1, p2, g1, g2, hb, vd, fr, nr = _routing(queries, w_sel)
    p1f, p2f = p1.reshape(N), p2.reshape(N)
    xg, gg = _scatter(queries, p1f, p2f, g1, g2)
    hbf, vdf, frf = hb.reshape(NBLK), vd.reshape(NBLK), fr.reshape(NBLK)
    nkv, nnm, yg = _passes(hbf, vdf, frf, nr.reshape(1), xg, gg,
                           wq, bq.reshape(H, 1, E), wk, bk.reshape(H, 1, E),
                           wv, bv.reshape(H, 1, E), mem_kv,
                           mem_norm.reshape(H, 1, E), wo, bo.reshape(H, 1, D))
    out = _combine(yg, p1f, p2f)
    return out, nkv, nnm.reshape(H, E)
